# Initial kernel scaffold; baseline (speedup 1.0000x reference)
import jax
import jax.numpy as jnp
from jax.experimental import pallas as pl


def kernel(a, W, b_conv, b_rec, k):
    raise NotImplementedError("write your pallas kernel here")


# speedup vs baseline: 4.7529x; 4.7529x over previous
import functools

import jax
import jax.numpy as jnp
from jax import lax
from jax.experimental import pallas as pl
from jax.experimental.pallas import tpu as pltpu
from jax.experimental.pallas import tpu_sc as plsc

NPRE = 64
ND = 8
NN = 128
TLEN = 32768
NFLAT = NN * TLEN
TT = 2048
NTT = TLEN // TT
JB = 128
NJB = TLEN // JB
NW = 32
SHARD = NFLAT // NW
WIN = 16384
NB = 2048
F32 = jnp.float32
I32 = jnp.int32


def _conv_body(a0_ref, a1_ref, w_ref, b_ref, p_ref):
    window = jnp.concatenate([a0_ref[...], a1_ref[:, :ND]], axis=1)
    acc = jnp.zeros((NN, TT), F32)
    for dd in range(ND):
        acc = acc + lax.dot_general(
            w_ref[:, :, dd], window[:, dd:dd + TT],
            (((1,), (0,)), ((), ())), preferred_element_type=F32)
    logit = acc + b_ref[...]
    p_ref[...] = 1.0 / (1.0 + jnp.exp(-logit))


def _conv_sigmoid(a_ext, W, b_conv_col):
    return pl.pallas_call(
        _conv_body,
        grid=(NTT,),
        in_specs=[
            pl.BlockSpec((NPRE, TT), lambda i: (0, i)),
            pl.BlockSpec((NPRE, TT), lambda i: (0, i + 1)),
            pl.BlockSpec((NN, NPRE, ND), lambda i: (0, 0, 0)),
            pl.BlockSpec((NN, 1), lambda i: (0, 0)),
        ],
        out_specs=pl.BlockSpec((NN, TT), lambda i: (0, i)),
        out_shape=jax.ShapeDtypeStruct((NN, TLEN), F32),
    )(a_ext, a_ext, W, b_conv_col)


def _make_hist_kernel(shift_d, mask_d, shift_p, use_pred):
    mesh = plsc.VectorSubcoreMesh(core_axis_name="c", subcore_axis_name="s")

    @functools.partial(
        pl.kernel, mesh=mesh,
        compiler_params=pltpu.CompilerParams(needs_layout_passes=False),
        out_type=jax.ShapeDtypeStruct((NW * NB * 16,), I32),
        scratch_types=[
            pltpu.VMEM((WIN,), I32),
            pltpu.VMEM((NB * 16,), I32),
            pltpu.VMEM((16,), I32),
        ],
    )
    def hist_kernel(u_hbm, prefix_hbm, out_hbm, buf, hist, pvec):
        wid = lax.axis_index("s") * 2 + lax.axis_index("c")
        base = wid * SHARD

        def zero_body(i, _):
            hist[pl.ds(i * 16, 16)] = jnp.zeros((16,), I32)
            return 0
        lax.fori_loop(0, NB, zero_body, 0)

        pltpu.sync_copy(prefix_hbm, pvec)
        pv = pvec[...]
        iota = lax.broadcasted_iota(I32, (16,), 0)
        ones = jnp.ones((16,), I32)

        for win in range(SHARD // WIN):
            pltpu.sync_copy(u_hbm.at[pl.ds(base + win * WIN, WIN)], buf)

            def body(j, _):
                u = buf[pl.ds(j * 16, 16)]
                digit = jnp.right_shift(u, shift_d) & mask_d
                idx = digit * 16 + iota
                if use_pred:
                    pred = jnp.right_shift(u, shift_p) == pv
                    plsc.addupdate_scatter(hist, [idx], ones, mask=pred)
                else:
                    plsc.addupdate_scatter(hist, [idx], ones)
                return 0
            lax.fori_loop(0, WIN // 16, body, 0)

        pltpu.sync_copy(hist, out_hbm.at[pl.ds(wid * (NB * 16), NB * 16)])

    return hist_kernel


_HIST_R0 = None
_HIST_R1 = None
_HIST_R2 = None


def _hist_kernels():
    global _HIST_R0, _HIST_R1, _HIST_R2
    if _HIST_R0 is None:
        _HIST_R0 = _make_hist_kernel(20, 0x7FF, 0, False)
        _HIST_R1 = _make_hist_kernel(9, 0x7FF, 20, True)
        _HIST_R2 = _make_hist_kernel(0, 0x1FF, 9, True)
    return _HIST_R0, _HIST_R1, _HIST_R2


def _merge_body(pshift, hist_ref, p_ref, c_ref, k_ref,
                prefout_ref, cout_ref, vstar_ref, m_ref):
    h = hist_ref[...].astype(F32)
    G = jnp.sum(jnp.sum(h, axis=3), axis=0)
    i0 = lax.broadcasted_iota(I32, (128, 128), 0)
    i1 = lax.broadcasted_iota(I32, (128, 128), 1)
    m128_gt = (i0 > i1).astype(F32)
    s_in = lax.dot_general(G, m128_gt, (((1,), (0,)), ((), ())),
                           preferred_element_type=F32)
    rt = jnp.sum(G, axis=1).reshape(1, 16)
    j0 = lax.broadcasted_iota(I32, (16, 16), 0)
    j1 = lax.broadcasted_iota(I32, (16, 16), 1)
    m16_gt = (j0 > j1).astype(F32)
    ra = lax.dot_general(rt, m16_gt, (((1,), (0,)), ((), ())),
                         preferred_element_type=F32)
    suf = s_in + ra.reshape(16, 1)
    c_prev = c_ref[0, 0]
    kk = k_ref[0, 0]
    tot = c_prev + suf
    cond = (tot < kk) & (tot + G >= kk)
    b0 = lax.broadcasted_iota(I32, (16, 128), 0)
    b1 = lax.broadcasted_iota(I32, (16, 128), 1)
    bid = b0 * 128 + b1
    dstar = jnp.sum(jnp.where(cond, bid, 0))
    new_c = c_prev + jnp.sum(jnp.where(cond, suf, 0.0))
    new_p = p_ref[0, 0] * (1 << pshift) + dstar
    prefout_ref[...] = jnp.full((1, 16), new_p, I32)
    cout_ref[0, 0] = new_c
    vstar_ref[0, 0] = lax.bitcast_convert_type(new_p, F32)
    m_ref[0, 0] = kk - new_c


def _merge(hist3, p_prev, c_prev, kf, pshift):
    return pl.pallas_call(
        functools.partial(_merge_body, pshift),
        in_specs=[
            pl.BlockSpec(memory_space=pltpu.VMEM),
            pl.BlockSpec(memory_space=pltpu.SMEM),
            pl.BlockSpec(memory_space=pltpu.SMEM),
            pl.BlockSpec(memory_space=pltpu.SMEM),
        ],
        out_specs=[
            pl.BlockSpec(memory_space=pltpu.VMEM),
            pl.BlockSpec(memory_space=pltpu.SMEM),
            pl.BlockSpec(memory_space=pltpu.SMEM),
            pl.BlockSpec(memory_space=pltpu.SMEM),
        ],
        out_shape=[
            jax.ShapeDtypeStruct((1, 16), I32),
            jax.ShapeDtypeStruct((1, 1), F32),
            jax.ShapeDtypeStruct((1, 1), F32),
            jax.ShapeDtypeStruct((1, 1), F32),
        ],
    )(hist3, p_prev, c_prev, kf)


def _select_vstar(u_flat, kf):
    h_r0, h_r1, h_r2 = _hist_kernels()
    zeros16 = jnp.zeros((16,), I32)
    p0 = jnp.zeros((1, 1), I32)
    c0 = jnp.zeros((1, 1), F32)

    hshape = (NW, 16, 128, 16)
    h0 = h_r0(u_flat, zeros16).reshape(hshape)
    pr1, c1, _, _ = _merge(h0, p0, c0, kf, 0)
    h1 = h_r1(u_flat, pr1.reshape(16))
    pr2, c2, _, _ = _merge(h1.reshape(hshape), pr1[:, :1], c1, kf, 11)
    h2 = h_r2(u_flat, pr2.reshape(16))
    _, _, vstar, m = _merge(h2.reshape(hshape), pr2[:, :1], c2, kf, 9)
    return vstar, m


def _bcnt_body(p_ref, v_ref, out_ref):
    v = v_ref[0, 0]
    eqf = jnp.where(p_ref[...] == v, 1.0, 0.0)
    cnt = jnp.sum(eqf, axis=1, keepdims=True)
    out_ref[...] = cnt.reshape(1, NN, 1)


def _block_counts(p, vstar):
    return pl.pallas_call(
        _bcnt_body,
        grid=(NJB,),
        in_specs=[
            pl.BlockSpec((NN, JB), lambda j: (0, j)),
            pl.BlockSpec(memory_space=pltpu.SMEM),
        ],
        out_specs=pl.BlockSpec((1, NN, 1), lambda j: (j, 0, 0)),
        out_shape=jax.ShapeDtypeStruct((NJB, NN, 1), F32),
    )(p, vstar)


def _offsets_body(b_ref, out_ref):
    B = b_ref[...]
    i0 = lax.broadcasted_iota(I32, (NJB, NJB), 0)
    i1 = lax.broadcasted_iota(I32, (NJB, NJB), 1)
    m_lt = (i1 < i0).astype(F32)
    o_in = lax.dot_general(m_lt, B, (((1,), (0,)), ((), ())),
                           preferred_element_type=F32)
    rt = jnp.sum(B, axis=0).reshape(1, NN)
    r0 = lax.broadcasted_iota(I32, (NN, NN), 0)
    r1 = lax.broadcasted_iota(I32, (NN, NN), 1)
    m_row = (r0 < r1).astype(F32)
    rp = lax.dot_general(rt, m_row, (((1,), (0,)), ((), ())),
                         preferred_element_type=F32)
    out_ref[...] = o_in + rp


def _offsets(bcnt):
    return pl.pallas_call(
        _offsets_body,
        out_shape=jax.ShapeDtypeStruct((NJB, NN), F32),
    )(bcnt)


def _spike_body(p_ref, o_ref, v_ref, m_ref, out_ref):
    v = v_ref[0, 0]
    mm = m_ref[0, 0]
    pt = p_ref[...]
    eq = pt == v
    gt = pt > v
    eqf = jnp.where(eq, 1.0, 0.0)
    c0 = lax.broadcasted_iota(I32, (JB, JB), 0)
    c1 = lax.broadcasted_iota(I32, (JB, JB), 1)
    m_lt = (c0 < c1).astype(F32)
    pre = lax.dot_general(eqf, m_lt, (((1,), (0,)), ((), ())),
                          preferred_element_type=F32)
    ocol = o_ref[...].reshape(NN, 1)
    rank = ocol + pre
    out_ref[...] = jnp.where(gt | (eq & (rank < mm)), 1.0, 0.0)


def _spike_raster(p, off3, vstar, m):
    return pl.pallas_call(
        _spike_body,
        grid=(NJB,),
        in_specs=[
            pl.BlockSpec((NN, JB), lambda j: (0, j)),
            pl.BlockSpec((1, NN, 1), lambda j: (j, 0, 0)),
            pl.BlockSpec(memory_space=pltpu.SMEM),
            pl.BlockSpec(memory_space=pltpu.SMEM),
        ],
        out_specs=pl.BlockSpec((NN, JB), lambda j: (0, j)),
        out_shape=jax.ShapeDtypeStruct((NN, TLEN), F32),
    )(p, off3, vstar, m)


def _est_body(s0_ref, s1_ref, sp0_ref, w_ref, b_ref, out_ref):
    window = jnp.concatenate([s0_ref[...], s1_ref[:, :ND]], axis=1)
    acc = jnp.zeros((NPRE, TT), F32)
    for sft in range(1, ND + 1):
        acc = acc + lax.dot_general(
            w_ref[:, :, ND - sft], window[:, sft:sft + TT],
            (((0,), (0,)), ((), ())), preferred_element_type=F32)
    est = acc + b_ref[...]
    corr = lax.dot_general(w_ref[:, :, 0], sp0_ref[...],
                           (((0,), (0,)), ((), ())),
                           preferred_element_type=F32) + b_ref[...]
    is_last = pl.program_id(0) == NTT - 1
    colmask = lax.broadcasted_iota(I32, (NPRE, TT), 1) == TT - 1
    out_ref[...] = jnp.where(is_last & colmask, corr, est)


def _est(spikes_ext, spikes0, W, b_rec_col):
    return pl.pallas_call(
        _est_body,
        grid=(NTT,),
        in_specs=[
            pl.BlockSpec((NN, TT), lambda i: (0, i)),
            pl.BlockSpec((NN, TT), lambda i: (0, i + 1)),
            pl.BlockSpec((NN, 1), lambda i: (0, 0)),
            pl.BlockSpec((NN, NPRE, ND), lambda i: (0, 0, 0)),
            pl.BlockSpec((NPRE, 1), lambda i: (0, 0)),
        ],
        out_specs=pl.BlockSpec((NPRE, TT), lambda i: (0, i)),
        out_shape=jax.ShapeDtypeStruct((NPRE, TLEN), F32),
    )(spikes_ext, spikes_ext, spikes0, W, b_rec_col)


def kernel(a, W, b_conv, b_rec, k):
    a = a.astype(F32)
    W = W.astype(F32)
    kf = jnp.asarray(k, F32).reshape(1, 1)

    a_ext = jnp.pad(a, ((0, 0), (ND, TT - ND)))
    p = _conv_sigmoid(a_ext, W, b_conv.reshape(NN, 1))

    u_flat = lax.bitcast_convert_type(p, I32).reshape(NFLAT)
    vstar, m = _select_vstar(u_flat, kf)

    bcnt = _block_counts(p, vstar).reshape(NJB, NN)
    off = _offsets(bcnt)
    spikes = _spike_raster(p, off.reshape(NJB, NN, 1), vstar, m)

    spikes_ext = jnp.pad(spikes, ((0, 0), (0, TT)))
    est = _est(spikes_ext, spikes[:, :1], W, b_rec.reshape(NPRE, 1))
    return (p, spikes, est)


# speedup vs baseline: 5.2225x; 1.0988x over previous
import functools

import jax
import jax.numpy as jnp
from jax import lax
from jax.experimental import pallas as pl
from jax.experimental.pallas import tpu as pltpu
from jax.experimental.pallas import tpu_sc as plsc

NPRE = 64
ND = 8
NN = 128
TLEN = 32768
NFLAT = NN * TLEN
TT = 2048
NTT = TLEN // TT
JB = 128
NJB = TLEN // JB
NW = 32
SHARD = NFLAT // NW
WIN = 16384
NB = 2048
F32 = jnp.float32
I32 = jnp.int32


def _conv_body(a0_ref, a1_ref, w_ref, b_ref, p_ref, u_ref):
    window = jnp.concatenate([a0_ref[...], a1_ref[:, :ND]], axis=1)
    acc = jnp.zeros((NN, TT), F32)
    for dd in range(ND):
        acc = acc + lax.dot_general(
            w_ref[:, :, dd], window[:, dd:dd + TT],
            (((1,), (0,)), ((), ())), preferred_element_type=F32)
    logit = acc + b_ref[...]
    p = 1.0 / (1.0 + jnp.exp(-logit))
    p_ref[...] = p
    u_ref[...] = lax.bitcast_convert_type(p, I32)


def _conv_sigmoid(a_ext, W, b_conv_col):
    return pl.pallas_call(
        _conv_body,
        grid=(NTT,),
        in_specs=[
            pl.BlockSpec((NPRE, TT), lambda i: (0, i)),
            pl.BlockSpec((NPRE, TT), lambda i: (0, i + 1)),
            pl.BlockSpec((NN, NPRE, ND), lambda i: (0, 0, 0)),
            pl.BlockSpec((NN, 1), lambda i: (0, 0)),
        ],
        out_specs=[
            pl.BlockSpec((NN, TT), lambda i: (0, i)),
            pl.BlockSpec((NN, TT), lambda i: (0, i)),
        ],
        out_shape=[
            jax.ShapeDtypeStruct((NN, TLEN), F32),
            jax.ShapeDtypeStruct((NN, TLEN), I32),
        ],
    )(a_ext, a_ext, W, b_conv_col)


def _make_hist_kernel(shift_d, mask_d, shift_p, use_pred):
    mesh = plsc.VectorSubcoreMesh(core_axis_name="c", subcore_axis_name="s")

    @functools.partial(
        pl.kernel, mesh=mesh,
        compiler_params=pltpu.CompilerParams(needs_layout_passes=False),
        out_type=jax.ShapeDtypeStruct((NW * NB * 16,), I32),
        scratch_types=[
            pltpu.VMEM((WIN,), I32),
            pltpu.VMEM((WIN,), I32),
            pltpu.VMEM((NB * 16,), I32),
            pltpu.VMEM((16,), I32),
            pltpu.SemaphoreType.DMA,
            pltpu.SemaphoreType.DMA,
        ],
    )
    def hist_kernel(u_hbm, prefix_hbm, out_hbm, buf0, buf1, hist, pvec,
                    sem0, sem1):
        wid = lax.axis_index("s") * 2 + lax.axis_index("c")
        base = wid * SHARD

        def zero_body(i, _):
            hist[pl.ds(i * 16, 16)] = jnp.zeros((16,), I32)
            return 0
        lax.fori_loop(0, NB, zero_body, 0, unroll=8)

        pltpu.sync_copy(prefix_hbm, pvec)
        pv = pvec[...]
        iota = lax.broadcasted_iota(I32, (16,), 0)
        ones = jnp.ones((16,), I32)
        bufs = (buf0, buf1)
        sems = (sem0, sem1)
        nwin = SHARD // WIN

        def make_body(buf):
            def body(j, _):
                u = buf[pl.ds(j * 16, 16)]
                if shift_d:
                    digit = jnp.right_shift(u, shift_d) & mask_d
                else:
                    digit = u & mask_d
                idx = digit * 16 + iota
                if use_pred:
                    pred = jnp.right_shift(u, shift_p) == pv
                    plsc.addupdate_scatter(hist, [idx], ones, mask=pred)
                else:
                    plsc.addupdate_scatter(hist, [idx], ones)
                return 0
            return body

        pending = pltpu.async_copy(u_hbm.at[pl.ds(base, WIN)], buf0, sem0)
        for w in range(nwin):
            cur = pending
            if w + 1 < nwin:
                pending = pltpu.async_copy(
                    u_hbm.at[pl.ds(base + (w + 1) * WIN, WIN)],
                    bufs[(w + 1) % 2], sems[(w + 1) % 2])
            cur.wait()
            lax.fori_loop(0, WIN // 16, make_body(bufs[w % 2]), 0, unroll=8)

        pltpu.sync_copy(hist, out_hbm.at[pl.ds(wid * (NB * 16), NB * 16)])

    return hist_kernel


_HIST_R0 = None
_HIST_R1 = None
_HIST_R2 = None


def _hist_kernels():
    global _HIST_R0, _HIST_R1, _HIST_R2
    if _HIST_R0 is None:
        _HIST_R0 = _make_hist_kernel(20, 0x7FF, 0, False)
        _HIST_R1 = _make_hist_kernel(9, 0x7FF, 20, True)
        _HIST_R2 = _make_hist_kernel(0, 0x1FF, 9, True)
    return _HIST_R0, _HIST_R1, _HIST_R2


def _merge_body(pshift, hist_ref, p_ref, c_ref, k_ref,
                prefout_ref, cout_ref, vstar_ref, m_ref):
    h = hist_ref[...].astype(F32)
    G = jnp.sum(jnp.sum(h, axis=3), axis=0)
    i0 = lax.broadcasted_iota(I32, (128, 128), 0)
    i1 = lax.broadcasted_iota(I32, (128, 128), 1)
    m128_gt = (i0 > i1).astype(F32)
    s_in = lax.dot_general(G, m128_gt, (((1,), (0,)), ((), ())),
                           preferred_element_type=F32)
    rt = jnp.sum(G, axis=1).reshape(1, 16)
    j0 = lax.broadcasted_iota(I32, (16, 16), 0)
    j1 = lax.broadcasted_iota(I32, (16, 16), 1)
    m16_gt = (j0 > j1).astype(F32)
    ra = lax.dot_general(rt, m16_gt, (((1,), (0,)), ((), ())),
                         preferred_element_type=F32)
    suf = s_in + ra.reshape(16, 1)
    c_prev = c_ref[0, 0]
    kk = k_ref[0, 0]
    tot = c_prev + suf
    cond = (tot < kk) & (tot + G >= kk)
    b0 = lax.broadcasted_iota(I32, (16, 128), 0)
    b1 = lax.broadcasted_iota(I32, (16, 128), 1)
    bid = b0 * 128 + b1
    dstar = jnp.sum(jnp.where(cond, bid, 0))
    new_c = c_prev + jnp.sum(jnp.where(cond, suf, 0.0))
    new_p = p_ref[0, 0] * (1 << pshift) + dstar
    prefout_ref[...] = jnp.full((1, 16), new_p, I32)
    cout_ref[0, 0] = new_c
    vstar_ref[0, 0] = lax.bitcast_convert_type(new_p, F32)
    m_ref[0, 0] = kk - new_c


def _merge(hist3, p_prev, c_prev, kf, pshift):
    return pl.pallas_call(
        functools.partial(_merge_body, pshift),
        in_specs=[
            pl.BlockSpec(memory_space=pltpu.VMEM),
            pl.BlockSpec(memory_space=pltpu.SMEM),
            pl.BlockSpec(memory_space=pltpu.SMEM),
            pl.BlockSpec(memory_space=pltpu.SMEM),
        ],
        out_specs=[
            pl.BlockSpec(memory_space=pltpu.VMEM),
            pl.BlockSpec(memory_space=pltpu.SMEM),
            pl.BlockSpec(memory_space=pltpu.SMEM),
            pl.BlockSpec(memory_space=pltpu.SMEM),
        ],
        out_shape=[
            jax.ShapeDtypeStruct((1, 16), I32),
            jax.ShapeDtypeStruct((1, 1), F32),
            jax.ShapeDtypeStruct((1, 1), F32),
            jax.ShapeDtypeStruct((1, 1), F32),
        ],
    )(hist3, p_prev, c_prev, kf)


def _select_vstar(u_flat, kf):
    h_r0, h_r1, h_r2 = _hist_kernels()
    zeros16 = jnp.zeros((16,), I32)
    p0 = jnp.zeros((1, 1), I32)
    c0 = jnp.zeros((1, 1), F32)

    hshape = (NW, 16, 128, 16)
    h0 = h_r0(u_flat, zeros16).reshape(hshape)
    pr1, c1, _, _ = _merge(h0, p0, c0, kf, 0)
    h1 = h_r1(u_flat, pr1.reshape(16))
    pr2, c2, _, _ = _merge(h1.reshape(hshape), pr1[:, :1], c1, kf, 11)
    h2 = h_r2(u_flat, pr2.reshape(16))
    _, _, vstar, m = _merge(h2.reshape(hshape), pr2[:, :1], c2, kf, 9)
    return vstar, m


def _bcnt_body(p_ref, v_ref, out_ref):
    v = v_ref[0, 0]
    eqf = jnp.where(p_ref[...] == v, 1.0, 0.0)
    cnt = jnp.sum(eqf, axis=1, keepdims=True)
    out_ref[...] = cnt.reshape(1, NN, 1)


def _block_counts(p, vstar):
    return pl.pallas_call(
        _bcnt_body,
        grid=(NJB,),
        in_specs=[
            pl.BlockSpec((NN, JB), lambda j: (0, j)),
            pl.BlockSpec(memory_space=pltpu.SMEM),
        ],
        out_specs=pl.BlockSpec((1, NN, 1), lambda j: (j, 0, 0)),
        out_shape=jax.ShapeDtypeStruct((NJB, NN, 1), F32),
    )(p, vstar)


def _offsets_body(b_ref, out_ref):
    B = b_ref[...]
    i0 = lax.broadcasted_iota(I32, (NJB, NJB), 0)
    i1 = lax.broadcasted_iota(I32, (NJB, NJB), 1)
    m_lt = (i1 < i0).astype(F32)
    o_in = lax.dot_general(m_lt, B, (((1,), (0,)), ((), ())),
                           preferred_element_type=F32)
    rt = jnp.sum(B, axis=0).reshape(1, NN)
    r0 = lax.broadcasted_iota(I32, (NN, NN), 0)
    r1 = lax.broadcasted_iota(I32, (NN, NN), 1)
    m_row = (r0 < r1).astype(F32)
    rp = lax.dot_general(rt, m_row, (((1,), (0,)), ((), ())),
                         preferred_element_type=F32)
    out_ref[...] = o_in + rp


def _offsets(bcnt):
    return pl.pallas_call(
        _offsets_body,
        out_shape=jax.ShapeDtypeStruct((NJB, NN), F32),
    )(bcnt)


def _spike_body(p_ref, o_ref, v_ref, m_ref, out_ref):
    v = v_ref[0, 0]
    mm = m_ref[0, 0]
    pt = p_ref[...]
    eq = pt == v
    gt = pt > v
    eqf = jnp.where(eq, 1.0, 0.0)
    c0 = lax.broadcasted_iota(I32, (JB, JB), 0)
    c1 = lax.broadcasted_iota(I32, (JB, JB), 1)
    m_lt = (c0 < c1).astype(F32)
    pre = lax.dot_general(eqf, m_lt, (((1,), (0,)), ((), ())),
                          preferred_element_type=F32)
    ocol = o_ref[...].reshape(NN, 1)
    rank = ocol + pre
    out_ref[...] = jnp.where(gt | (eq & (rank < mm)), 1.0, 0.0)


def _spike_raster(p, off3, vstar, m):
    return pl.pallas_call(
        _spike_body,
        grid=(NJB,),
        in_specs=[
            pl.BlockSpec((NN, JB), lambda j: (0, j)),
            pl.BlockSpec((1, NN, 1), lambda j: (j, 0, 0)),
            pl.BlockSpec(memory_space=pltpu.SMEM),
            pl.BlockSpec(memory_space=pltpu.SMEM),
        ],
        out_specs=pl.BlockSpec((NN, JB), lambda j: (0, j)),
        out_shape=jax.ShapeDtypeStruct((NN, TLEN), F32),
    )(p, off3, vstar, m)


def _est_body(s0_ref, s1_ref, sp0_ref, w_ref, b_ref, out_ref):
    is_last = pl.program_id(0) == NTT - 1
    nxt = jnp.where(is_last, 0.0, 1.0) * s1_ref[:, :ND]
    window = jnp.concatenate([s0_ref[...], nxt], axis=1)
    acc = jnp.zeros((NPRE, TT), F32)
    for sft in range(1, ND + 1):
        acc = acc + lax.dot_general(
            w_ref[:, :, ND - sft], window[:, sft:sft + TT],
            (((0,), (0,)), ((), ())), preferred_element_type=F32)
    est = acc + b_ref[...]
    corr = lax.dot_general(w_ref[:, :, 0], sp0_ref[...],
                           (((0,), (0,)), ((), ())),
                           preferred_element_type=F32) + b_ref[...]
    colmask = lax.broadcasted_iota(I32, (NPRE, TT), 1) == TT - 1
    out_ref[...] = jnp.where(is_last & colmask, corr, est)


def _est(spikes, spikes0, W, b_rec_col):
    return pl.pallas_call(
        _est_body,
        grid=(NTT,),
        in_specs=[
            pl.BlockSpec((NN, TT), lambda i: (0, i)),
            pl.BlockSpec((NN, TT), lambda i: (0, jnp.minimum(i + 1, NTT - 1))),
            pl.BlockSpec((NN, 1), lambda i: (0, 0)),
            pl.BlockSpec((NN, NPRE, ND), lambda i: (0, 0, 0)),
            pl.BlockSpec((NPRE, 1), lambda i: (0, 0)),
        ],
        out_specs=pl.BlockSpec((NPRE, TT), lambda i: (0, i)),
        out_shape=jax.ShapeDtypeStruct((NPRE, TLEN), F32),
    )(spikes, spikes, spikes0, W, b_rec_col)


def kernel(a, W, b_conv, b_rec, k):
    a = a.astype(F32)
    W = W.astype(F32)
    kf = jnp.asarray(k, F32).reshape(1, 1)

    a_ext = jnp.pad(a, ((0, 0), (ND, TT - ND)))
    p, u = _conv_sigmoid(a_ext, W, b_conv.reshape(NN, 1))

    vstar, m = _select_vstar(u.reshape(NFLAT), kf)

    bcnt = _block_counts(p, vstar).reshape(NJB, NN)
    off = _offsets(bcnt)
    spikes = _spike_raster(p, off.reshape(NJB, NN, 1), vstar, m)

    est = _est(spikes, spikes[:, :1], W, b_rec.reshape(NPRE, 1))
    return (p, spikes, est)


# speedup vs baseline: 10.0117x; 1.9170x over previous
import functools

import jax
import jax.numpy as jnp
from jax import lax
from jax.experimental import pallas as pl
from jax.experimental.pallas import tpu as pltpu
from jax.experimental.pallas import tpu_sc as plsc

NPRE = 64
ND = 8
NN = 128
TLEN = 32768
NFLAT = NN * TLEN
TTC = 8192
NTC = TLEN // TTC
TT = 2048
NTT = TLEN // TT
NW = 32
SHARD = NFLAT // NW
WIN = 8192
NB = 2048
HC = 2
HTOT = HC * NB * 16
F32 = jnp.float32
I32 = jnp.int32


def _iota2(shape, dim):
    return lax.broadcasted_iota(I32, shape, dim)


def _conv_body(a0_ref, a1_ref, w_ref, b_ref, p_ref, u_ref):
    window = jnp.concatenate([a0_ref[...], a1_ref[:, :ND]], axis=1)
    acc = jnp.zeros((NN, TTC), F32)
    for dd in range(ND):
        acc = acc + lax.dot_general(
            w_ref[:, :, dd], window[:, dd:dd + TTC],
            (((1,), (0,)), ((), ())), preferred_element_type=F32)
    logit = acc + b_ref[...]
    p = 1.0 / (1.0 + jnp.exp(-logit))
    p_ref[...] = p
    u_ref[...] = lax.bitcast_convert_type(p, I32)


def _conv_sigmoid(a_ext, W, b_conv_col):
    return pl.pallas_call(
        _conv_body,
        grid=(NTC,),
        in_specs=[
            pl.BlockSpec((NPRE, TTC), lambda i: (0, i)),
            pl.BlockSpec((NPRE, TTC), lambda i: (0, i + 1)),
            pl.BlockSpec((NN, NPRE, ND), lambda i: (0, 0, 0)),
            pl.BlockSpec((NN, 1), lambda i: (0, 0)),
        ],
        out_specs=[
            pl.BlockSpec((NN, TTC), lambda i: (0, i)),
            pl.BlockSpec((NN, TTC), lambda i: (0, i)),
        ],
        out_shape=[
            jax.ShapeDtypeStruct((NN, TLEN), F32),
            jax.ShapeDtypeStruct((NN, TLEN), I32),
        ],
    )(a_ext, a_ext, W, b_conv_col)


def _make_hist_kernel(shift_d, mask_d, shift_p, use_pred):
    mesh = plsc.VectorSubcoreMesh(core_axis_name="c", subcore_axis_name="s")

    @functools.partial(
        pl.kernel, mesh=mesh,
        compiler_params=pltpu.CompilerParams(needs_layout_passes=False),
        out_type=jax.ShapeDtypeStruct((NW * HTOT,), I32),
        scratch_types=[
            pltpu.VMEM((WIN,), I32),
            pltpu.VMEM((WIN,), I32),
            pltpu.VMEM((HTOT,), I32),
            pltpu.VMEM((16,), I32),
            pltpu.SemaphoreType.DMA,
            pltpu.SemaphoreType.DMA,
        ],
    )
    def hist_kernel(u_hbm, prefix_hbm, out_hbm, buf0, buf1, hist, pvec,
                    sem0, sem1):
        wid = lax.axis_index("s") * 2 + lax.axis_index("c")
        base = wid * SHARD

        def zero_body(i, _):
            hist[pl.ds(i * 16, 16)] = jnp.zeros((16,), I32)
            return 0
        lax.fori_loop(0, HTOT // 16, zero_body, 0, unroll=8)

        pltpu.sync_copy(prefix_hbm, pvec)
        pv = pvec[...]
        iota = lax.broadcasted_iota(I32, (16,), 0)
        ones = jnp.ones((16,), I32)
        bufs = (buf0, buf1)
        sems = (sem0, sem1)
        nwin = SHARD // WIN

        def make_body(buf):
            def body(j, _):
                for k in range(HC):
                    u = buf[pl.ds((j * HC + k) * 16, 16)]
                    if shift_d:
                        digit = jnp.right_shift(u, shift_d) & mask_d
                    else:
                        digit = u & mask_d
                    idx = (k * (NB * 16) + digit * 16) + iota
                    if use_pred:
                        pred = jnp.right_shift(u, shift_p) == pv
                        plsc.addupdate_scatter(hist, [idx], ones, mask=pred)
                    else:
                        plsc.addupdate_scatter(hist, [idx], ones)
                return 0
            return body

        pending = pltpu.async_copy(u_hbm.at[pl.ds(base, WIN)], buf0, sem0)
        for w in range(nwin):
            cur = pending
            if w + 1 < nwin:
                pending = pltpu.async_copy(
                    u_hbm.at[pl.ds(base + (w + 1) * WIN, WIN)],
                    bufs[(w + 1) % 2], sems[(w + 1) % 2])
            cur.wait()
            lax.fori_loop(0, WIN // (16 * HC), make_body(bufs[w % 2]), 0,
                          unroll=4)

        pltpu.sync_copy(hist, out_hbm.at[pl.ds(wid * HTOT, HTOT)])

    return hist_kernel


_HIST_KERNELS = None


def _hist_kernels():
    global _HIST_KERNELS
    if _HIST_KERNELS is None:
        _HIST_KERNELS = (_make_hist_kernel(20, 0x7FF, 0, False),
                         _make_hist_kernel(9, 0x7FF, 20, True),
                         _make_hist_kernel(0, 0x1FF, 9, True))
    return _HIST_KERNELS


def _merge_body(pshift, hist_ref, p_ref, c_ref, k_ref,
                prefout_ref, cout_ref, vstar_ref, m_ref):
    h = hist_ref[...]
    hw = jnp.sum(h, axis=0)
    h2 = hw.reshape(HTOT // 128, 128).astype(F32)
    l0 = _iota2((128, 8), 0)
    l1 = _iota2((128, 8), 1)
    g16 = ((l0 // 16) == l1).astype(F32)
    gc = lax.dot_general(h2, g16, (((1,), (0,)), ((), ())),
                         preferred_element_type=F32)
    g = jnp.sum(gc.reshape(HC, NB // 8, 8), axis=0)
    ones8 = jnp.ones((8, 1), F32)
    rt = lax.dot_general(g, ones8, (((1,), (0,)), ((), ())),
                         preferred_element_type=F32)
    r0 = _iota2((NB // 8, NB // 8), 0)
    r1 = _iota2((NB // 8, NB // 8), 1)
    mgt = (r1 > r0).astype(F32)
    ra = lax.dot_general(mgt, rt, (((1,), (0,)), ((), ())),
                         preferred_element_type=F32)
    c0 = _iota2((8, 8), 0)
    c1 = _iota2((8, 8), 1)
    m8gt = (c0 > c1).astype(F32)
    s_in = lax.dot_general(g, m8gt, (((1,), (0,)), ((), ())),
                           preferred_element_type=F32)
    suf = s_in + ra
    c_prev = c_ref[0, 0]
    kk = k_ref[0, 0]
    tot = c_prev + suf
    cond = (tot < kk) & (tot + g >= kk)
    bid = _iota2((NB // 8, 8), 0) * 8 + _iota2((NB // 8, 8), 1)
    dstar = jnp.sum(jnp.where(cond, bid, 0))
    new_c = c_prev + jnp.sum(jnp.where(cond, suf, 0.0))
    new_p = p_ref[0, 0] * (1 << pshift) + dstar
    prefout_ref[...] = jnp.full((1, 16), new_p, I32)
    cout_ref[0, 0] = new_c
    vstar_ref[0, 0] = lax.bitcast_convert_type(new_p, F32)
    m_ref[0, 0] = kk - new_c


def _merge(hist2, p_prev, c_prev, kf, pshift):
    return pl.pallas_call(
        functools.partial(_merge_body, pshift),
        in_specs=[
            pl.BlockSpec(memory_space=pltpu.VMEM),
            pl.BlockSpec(memory_space=pltpu.SMEM),
            pl.BlockSpec(memory_space=pltpu.SMEM),
            pl.BlockSpec(memory_space=pltpu.SMEM),
        ],
        out_specs=[
            pl.BlockSpec(memory_space=pltpu.VMEM),
            pl.BlockSpec(memory_space=pltpu.SMEM),
            pl.BlockSpec(memory_space=pltpu.SMEM),
            pl.BlockSpec(memory_space=pltpu.SMEM),
        ],
        out_shape=[
            jax.ShapeDtypeStruct((1, 16), I32),
            jax.ShapeDtypeStruct((1, 1), F32),
            jax.ShapeDtypeStruct((1, 1), F32),
            jax.ShapeDtypeStruct((1, 1), F32),
        ],
    )(hist2, p_prev, c_prev, kf)


def _select_vstar(u_flat, kf):
    h_r0, h_r1, h_r2 = _hist_kernels()
    zeros16 = jnp.zeros((16,), I32)
    p0 = jnp.zeros((1, 1), I32)
    c0 = jnp.zeros((1, 1), F32)

    h0 = h_r0(u_flat, zeros16).reshape(NW, HTOT)
    pr1, c1, _, _ = _merge(h0, p0, c0, kf, 0)
    h1 = h_r1(u_flat, pr1.reshape(16))
    pr2, c2, _, _ = _merge(h1.reshape(NW, HTOT), pr1[:, :1], c1, kf, 11)
    h2 = h_r2(u_flat, pr2.reshape(16))
    _, _, vstar, m = _merge(h2.reshape(NW, HTOT), pr2[:, :1], c2, kf, 9)
    return vstar, m


def _bcnt_body(p_ref, v_ref, out_ref):
    v = v_ref[0, 0]
    eqf = jnp.where(p_ref[...] == v, 1.0, 0.0)
    cnt = jnp.sum(eqf, axis=1, keepdims=True)
    i0 = _iota2((NN, NN), 0)
    i1 = _iota2((NN, NN), 1)
    ident = (i0 == i1).astype(F32)
    cnt_l = lax.dot_general(cnt, ident, (((0,), (0,)), ((), ())),
                            preferred_element_type=F32)
    out_ref[...] = cnt_l.reshape(1, 1, NN)


def _block_counts(p, vstar):
    return pl.pallas_call(
        _bcnt_body,
        grid=(NTT,),
        in_specs=[
            pl.BlockSpec((NN, TT), lambda i: (0, i)),
            pl.BlockSpec(memory_space=pltpu.SMEM),
        ],
        out_specs=pl.BlockSpec((1, 1, NN), lambda i: (i, 0, 0)),
        out_shape=jax.ShapeDtypeStruct((NTT, 1, NN), F32),
    )(p, vstar)


def _offsets_body(b_ref, out_ref):
    B = b_ref[...].reshape(NTT, NN)
    t0 = _iota2((NTT, NTT), 0)
    t1 = _iota2((NTT, NTT), 1)
    m_lt = (t1 < t0).astype(F32)
    pre = lax.dot_general(m_lt, B, (((1,), (0,)), ((), ())),
                          preferred_element_type=F32)
    rt = jnp.sum(B, axis=0).reshape(1, NN)
    o0 = _iota2((NN, NN), 0)
    o1 = _iota2((NN, NN), 1)
    m_row = (o0 < o1).astype(F32)
    rp = lax.dot_general(rt, m_row, (((1,), (0,)), ((), ())),
                         preferred_element_type=F32)
    out_ref[...] = (pre + rp).reshape(NTT, 1, NN)


def _offsets(bcnt):
    return pl.pallas_call(
        _offsets_body,
        out_shape=jax.ShapeDtypeStruct((NTT, 1, NN), F32),
    )(bcnt)


def _spike_est_body(p0_ref, p1_ref, off_ref, off1_ref, off0_ref, pc0_ref,
                    w_ref, b_ref, v_ref, m_ref, sp_ref, est_ref):
    v = v_ref[0, 0]
    mm = m_ref[0, 0]
    i0 = _iota2((NN, NN), 0)
    i1 = _iota2((NN, NN), 1)
    ident = (i0 == i1).astype(F32)
    m_lt = (i0 < i1).astype(F32)

    def col_of(row):
        return lax.dot_general(ident, row, (((1,), (1,)), ((), ())),
                               preferred_element_type=F32)

    pt = p0_ref[...]
    eqf = jnp.where(pt == v, 1.0, 0.0)
    gtf = jnp.where(pt > v, 1.0, 0.0)
    run = col_of(off_ref[...].reshape(1, NN))
    pieces = []
    for jloc in range(TT // NN):
        sl = slice(jloc * NN, (jloc + 1) * NN)
        eqb = eqf[:, sl]
        pre = lax.dot_general(eqb, m_lt, (((1,), (0,)), ((), ())),
                              preferred_element_type=F32)
        rank = run + pre
        pieces.append(gtf[:, sl] + eqb * jnp.where(rank < mm, 1.0, 0.0))
        run = run + jnp.sum(eqb, axis=1, keepdims=True)
    spikes = jnp.concatenate(pieces, axis=1)
    sp_ref[...] = spikes

    is_last = pl.program_id(0) == NTT - 1
    p8 = p1_ref[:, :ND]
    eq8 = jnp.where(p8 == v, 1.0, 0.0)
    gt8 = jnp.where(p8 > v, 1.0, 0.0)
    c8a = _iota2((ND, ND), 0)
    c8b = _iota2((ND, ND), 1)
    m8_lt = (c8a < c8b).astype(F32)
    pre8 = lax.dot_general(eq8, m8_lt, (((1,), (0,)), ((), ())),
                           preferred_element_type=F32)
    rank8 = col_of(off1_ref[...].reshape(1, NN)) + pre8
    sp8 = gt8 + eq8 * jnp.where(rank8 < mm, 1.0, 0.0)
    sp8 = jnp.where(is_last, 0.0, 1.0) * sp8

    window = jnp.concatenate([spikes, sp8], axis=1)
    acc = jnp.zeros((NPRE, TT), F32)
    for sft in range(1, ND + 1):
        acc = acc + lax.dot_general(
            w_ref[:, :, ND - sft], window[:, sft:sft + TT],
            (((0,), (0,)), ((), ())), preferred_element_type=F32)
    est = acc + b_ref[...]

    p0c = pc0_ref[...]
    eq0 = jnp.where(p0c == v, 1.0, 0.0)
    gt0 = jnp.where(p0c > v, 1.0, 0.0)
    rank0 = col_of(off0_ref[...].reshape(1, NN))
    sp0 = gt0 + eq0 * jnp.where(rank0 < mm, 1.0, 0.0)
    corr = lax.dot_general(w_ref[:, :, 0], sp0, (((0,), (0,)), ((), ())),
                           preferred_element_type=F32) + b_ref[...]
    colmask = _iota2((NPRE, TT), 1) == TT - 1
    est_ref[...] = jnp.where(is_last & colmask, corr, est)


def _spike_est(p, off3, pcol0, W, b_rec_col, vstar, m):
    return pl.pallas_call(
        _spike_est_body,
        grid=(NTT,),
        in_specs=[
            pl.BlockSpec((NN, TT), lambda i: (0, i)),
            pl.BlockSpec((NN, TT), lambda i: (0, jnp.minimum(i + 1, NTT - 1))),
            pl.BlockSpec((1, 1, NN), lambda i: (i, 0, 0)),
            pl.BlockSpec((1, 1, NN),
                         lambda i: (jnp.minimum(i + 1, NTT - 1), 0, 0)),
            pl.BlockSpec((1, 1, NN), lambda i: (0, 0, 0)),
            pl.BlockSpec((NN, 1), lambda i: (0, 0)),
            pl.BlockSpec((NN, NPRE, ND), lambda i: (0, 0, 0)),
            pl.BlockSpec((NPRE, 1), lambda i: (0, 0)),
            pl.BlockSpec(memory_space=pltpu.SMEM),
            pl.BlockSpec(memory_space=pltpu.SMEM),
        ],
        out_specs=[
            pl.BlockSpec((NN, TT), lambda i: (0, i)),
            pl.BlockSpec((NPRE, TT), lambda i: (0, i)),
        ],
        out_shape=[
            jax.ShapeDtypeStruct((NN, TLEN), F32),
            jax.ShapeDtypeStruct((NPRE, TLEN), F32),
        ],
    )(p, p, off3, off3, off3, pcol0, W, b_rec_col, vstar, m)


def kernel(a, W, b_conv, b_rec, k):
    a = a.astype(F32)
    W = W.astype(F32)
    kf = jnp.asarray(k, F32).reshape(1, 1)

    a_ext = jnp.pad(a, ((0, 0), (ND, TTC - ND)))
    p, u = _conv_sigmoid(a_ext, W, b_conv.reshape(NN, 1))

    vstar, m = _select_vstar(u.reshape(NFLAT), kf)

    bcnt = _block_counts(p, vstar)
    off3 = _offsets(bcnt)
    spikes, est = _spike_est(p, off3, p[:, :1], W, b_rec.reshape(NPRE, 1),
                             vstar, m)
    return (p, spikes, est)
